# baseline (speedup 1.0000x reference)
import functools

import jax
import jax.numpy as jnp
import numpy as np
from jax import lax
from jax.experimental import pallas as pl
from jax.experimental.pallas import tpu as pltpu
from jax.experimental.pallas import tpu_sc as plsc

N = 10000
E = 160000
LANE = 128
E_PAD = 163840
IDX_ROWS = E_PAD // LANE
TILES = 16
ROWS_PER_TILE = IDX_ROWS // TILES
UNIT = 8
N_UNITS = ROWS_PER_TILE // UNIT
NBUF = 4
SLOT = 64
USLOTS = 2 * UNIT
SLOTS = N_UNITS * USLOTS
STRIPE = 624
EXTRA = N - TILES * STRIPE
JUNK = 8
BM = 2000
VEC = 16

_PAD_SRC = np.arange(E_PAD - E, dtype=np.int32) % N
_PAD_DST = N + np.arange(E_PAD - E, dtype=np.int32) % JUNK


def _make_segsum(n_chunks):
    half_ch = n_chunks // 2
    mesh = plsc.VectorSubcoreMesh(core_axis_name="c", subcore_axis_name="s")

    @functools.partial(
        pl.kernel,
        mesh=mesh,
        out_type=[jax.ShapeDtypeStruct((N, LANE), jnp.float32)] * n_chunks,
        scratch_types=[
            pltpu.VMEM((UNIT, LANE), jnp.int32),
            pltpu.VMEM((2, USLOTS, SLOT), jnp.int32),
            pltpu.VMEM((2, USLOTS, SLOT), jnp.int32),
            pltpu.VMEM((NBUF, SLOT, LANE), jnp.float32),
            pltpu.VMEM_SHARED((N + JUNK, LANE), jnp.float32),
            pltpu.SemaphoreType.DMA((NBUF,)),
            pltpu.SemaphoreType.DMA((NBUF,)),
        ],
    )
    def segsum(xf, src_r, dst_r, zero_r, *rest):
        outs = rest[:n_chunks]
        sbuf, sidx, didx, rows, acc, gsem, ssem = rest[n_chunks:]

        c = lax.axis_index("c")
        s = lax.axis_index("s")

        n_grp = SLOTS // NBUF

        def process(ch, out_ref):
            pltpu.sync_copy(zero_r, acc.at[pl.ds(s * STRIPE, STRIPE)])

            @pl.when(s == TILES - 1)
            def _():
                pltpu.sync_copy(
                    zero_r.at[pl.ds(0, EXTRA)],
                    acc.at[pl.ds(TILES * STRIPE, EXTRA)],
                )

            plsc.subcore_barrier()

            def gather(j, b):
                pltpu.async_copy(
                    xf.at[sidx.at[(j // USLOTS) % 2, j % USLOTS]],
                    rows.at[b], gsem.at[b],
                )

            def gather_wait(j, b):
                pltpu.make_async_copy(
                    xf.at[sidx.at[(j // USLOTS) % 2, j % USLOTS]],
                    rows.at[b], gsem.at[b],
                ).wait()

            def scatter(j, b):
                pltpu.async_copy(
                    rows.at[b],
                    acc.at[didx.at[(j // USLOTS) % 2, j % USLOTS]],
                    ssem.at[b], add=True,
                )

            def scatter_wait(j, b):
                pltpu.make_async_copy(
                    rows.at[b],
                    acc.at[didx.at[(j // USLOTS) % 2, j % USLOTS]],
                    ssem.at[b],
                ).wait()

            def stage(u):
                p = u % 2
                rb = s * ROWS_PER_TILE + u * UNIT
                pltpu.sync_copy(src_r.at[pl.ds(rb, UNIT)], sbuf)

                def idxt_s(r, carry):
                    for v in range(LANE // VEC):
                        dst_sl = pl.ds((v % 4) * VEC, VEC)
                        sidx[p, 2 * r + v // 4, dst_sl] = (
                            sbuf[r, pl.ds(v * VEC, VEC)] * n_chunks + ch
                        )
                    return carry

                lax.fori_loop(0, UNIT, idxt_s, 0)
                pltpu.sync_copy(dst_r.at[pl.ds(rb, UNIT)], sbuf)

                def idxt_d(r, carry):
                    for v in range(LANE // VEC):
                        dst_sl = pl.ds((v % 4) * VEC, VEC)
                        didx[p, 2 * r + v // 4, dst_sl] = sbuf[
                            r, pl.ds(v * VEC, VEC)
                        ]
                    return carry

                lax.fori_loop(0, UNIT, idxt_d, 0)

            stage(0)
            for b in range(NBUF - 1):
                gather(b, b)

            def grp(g, carry):
                j0 = NBUF * g
                for k in range(NBUF):
                    j = j0 + k
                    gather_wait(j, k)
                    scatter(j, k)

                    if k == 0:
                        @pl.when(g > 0)
                        def _():
                            scatter_wait(j0 - 1, NBUF - 1)

                        @pl.when(lax.rem(g, NBUF) == 0)
                        def _():
                            @pl.when(g < n_grp - NBUF)
                            def _():
                                stage(g // NBUF + 1)
                    else:
                        scatter_wait(j - 1, k - 1)

                    @pl.when(j < SLOTS - NBUF + 1)
                    def _(j=j, k=k):
                        gather(j + NBUF - 1, (k + NBUF - 1) % NBUF)

                return carry

            lax.fori_loop(0, n_grp, grp, 0)
            scatter_wait(SLOTS - 1, NBUF - 1)
            plsc.subcore_barrier()
            pltpu.sync_copy(
                acc.at[pl.ds(s * STRIPE, STRIPE)],
                out_ref.at[pl.ds(s * STRIPE, STRIPE)],
            )

            @pl.when(s == TILES - 1)
            def _():
                pltpu.sync_copy(
                    acc.at[pl.ds(TILES * STRIPE, EXTRA)],
                    out_ref.at[pl.ds(TILES * STRIPE, EXTRA)],
                )

        for cid in range(2):
            for j in range(half_ch):
                ch = cid * half_ch + j

                @pl.when(c == cid)
                def _(ch=ch):
                    process(ch, outs[ch])

    return segsum


def _fused_matmul(lhs_list, rhs_list, bias, residual, relu,
                  lhs_fold=None, out_fold=1):
    n_in = len(lhs_list)
    lhs_fold = lhs_fold or [1] * n_in
    m = lhs_list[0].shape[0] // lhs_fold[0]
    n_out = rhs_list[0].shape[1]
    grid = (m // BM,)
    has_res = residual is not None

    def body(*refs):
        ls = refs[:n_in]
        rs = refs[n_in : 2 * n_in]
        b = refs[2 * n_in]
        res = refs[2 * n_in + 1] if has_res else None
        out = refs[-1]
        acc = b[...]
        if has_res:
            acc = acc + res[...]
        for i in range(n_in):
            lv = ls[i][...]
            if lhs_fold[i] > 1:
                lv = lv.reshape(BM, LANE * lhs_fold[i])
            acc = acc + jnp.dot(
                lv, rs[i][...], preferred_element_type=jnp.float32
            )
        if relu:
            acc = jnp.maximum(acc, 0.0)
        if out_fold > 1:
            out[...] = acc.reshape(out_fold * BM, LANE)
        else:
            out[...] = acc

    in_specs = (
        [
            pl.BlockSpec((f * BM, l.shape[1]), lambda mi: (mi, 0))
            for l, f in zip(lhs_list, lhs_fold)
        ]
        + [pl.BlockSpec(r.shape, lambda mi: (0, 0)) for r in rhs_list]
        + [pl.BlockSpec((1, n_out), lambda mi: (0, 0))]
    )
    args = list(lhs_list) + list(rhs_list) + [bias.reshape(1, n_out)]
    if has_res:
        in_specs.append(pl.BlockSpec((BM, n_out), lambda mi: (mi, 0)))
        args.append(residual)
    if out_fold > 1:
        out_specs = pl.BlockSpec((out_fold * BM, LANE), lambda mi: (mi, 0))
        out_shape = jax.ShapeDtypeStruct((out_fold * m, LANE), jnp.float32)
    else:
        out_specs = pl.BlockSpec((BM, n_out), lambda mi: (mi, 0))
        out_shape = jax.ShapeDtypeStruct((m, n_out), jnp.float32)
    return pl.pallas_call(
        body,
        grid=grid,
        in_specs=in_specs,
        out_specs=out_specs,
        out_shape=out_shape,
    )(*args)


def kernel(x, edge_attr, W_rel1, b_rel1, W_root1, W_rel2, b_rel2, W_root2,
           edge_index):
    del edge_attr
    src = edge_index[0].astype(jnp.int32)
    dst = edge_index[1].astype(jnp.int32)
    src_p = jnp.concatenate([src, _PAD_SRC]).reshape(IDX_ROWS, LANE)
    dst_p = jnp.concatenate([dst, _PAD_DST]).reshape(IDX_ROWS, LANE)
    zeros = jnp.zeros((STRIPE, LANE), jnp.float32)

    r1 = _fused_matmul([x], [W_root1.T], b_rel1, None, relu=False)
    a0, a1 = _make_segsum(2)(x.reshape(2 * N, LANE), src_p, dst_p, zeros)
    Wr1 = W_rel1.T
    h4 = _fused_matmul([a0, a1], [Wr1[:LANE], Wr1[LANE:]], b_rel1 * 0.0, r1,
                       relu=True, out_fold=4)

    r2 = _fused_matmul([h4], [W_root2.T], b_rel2, None, relu=False,
                       lhs_fold=[4])
    b_chunks = _make_segsum(4)(h4, src_p, dst_p, zeros)
    Wr2 = W_rel2.T
    out = _fused_matmul(
        list(b_chunks),
        [Wr2[i * LANE : (i + 1) * LANE] for i in range(4)],
        b_rel2 * 0.0,
        r2,
        relu=False,
    )
    return out
